# baseline (speedup 1.0000x reference)
import functools

import jax
import jax.numpy as jnp
from jax import lax
from jax.experimental import pallas as pl
from jax.experimental.pallas import tpu as pltpu
from jax.experimental.pallas import tpu_sc as plsc

B = 16384
N = 50
D = 64
K = 64
OUT = 64

NC = 2
NS = 16
L = 16

WORDS = OUT // 2
ROWS_PER_TEC = B // (NC * NS)
CHUNK = 128
GROUPS = CHUNK // L
NCHUNK = ROWS_PER_TEC // CHUNK
PASSES = 4
WPP = WORDS // PASSES


def _tc_table(node_embeds, w, b2):
    def body(e_ref, w_ref, b_ref, o_ref):
        e = e_ref[...]
        bias = b_ref[...]
        for n in range(N):
            w_blk = w_ref[:, n * D:(n + 1) * D]
            p = lax.dot_general(e, w_blk, (((1,), (1,)), ((), ())),
                                preferred_element_type=jnp.float32)
            if n == 0:
                p = p + bias
            o_ref[pl.ds(n * K, K), :] = p.astype(jnp.bfloat16)

    return pl.pallas_call(
        body,
        out_shape=jax.ShapeDtypeStruct((N * K, OUT), jnp.bfloat16),
    )(node_embeds, w, b2)


_sc_mesh = plsc.VectorSubcoreMesh(
    core_axis_name="c", subcore_axis_name="s", num_cores=NC, num_subcores=NS)


@functools.partial(
    pl.kernel,
    out_type=jax.ShapeDtypeStruct((B * OUT,), jnp.float32),
    mesh=_sc_mesh,
    compiler_params=pltpu.CompilerParams(needs_layout_passes=False),
    scratch_types=[
        pltpu.VMEM((N * K * WORDS,), jnp.int32),
        pltpu.VMEM((N, CHUNK), jnp.int32),
        pltpu.VMEM((N, CHUNK), jnp.int32),
        pltpu.VMEM((CHUNK * OUT,), jnp.float32),
        pltpu.SemaphoreType.DMA,
        pltpu.SemaphoreType.DMA,
        pltpu.SemaphoreType.DMA,
        pltpu.SemaphoreType.DMA,
    ],
)
def _sc_gather_sum(tbl_hbm, invt_hbm, out_hbm,
                   tbl_v, inv_v0, inv_v1, outb_v,
                   tbl_sem, i_sem0, i_sem1, o_sem):
    cid = lax.axis_index("c")
    sid = lax.axis_index("s")
    wid = sid * NC + cid
    rowbase = wid * ROWS_PER_TEC
    inv_bufs = (inv_v0, inv_v1)
    i_sems = (i_sem0, i_sem1)

    tbl_cp = pltpu.async_copy(tbl_hbm, tbl_v, tbl_sem)

    def inv_copy(ch):
        row0 = rowbase + ch * CHUNK
        return pltpu.async_copy(
            invt_hbm.at[:, pl.ds(row0, CHUNK)], inv_bufs[ch % 2],
            i_sems[ch % 2])

    inv_cps = [inv_copy(0)]
    tbl_cp.wait()

    lane = lax.iota(jnp.int32, L)
    hi_mask = jnp.full((L,), -65536, jnp.int32)
    out_cp = None

    for ch in range(NCHUNK):
        buf = ch % 2
        inv_cps[ch].wait()
        if ch + 1 < NCHUNK:
            inv_cps.append(inv_copy(ch + 1))
        if out_cp is not None:
            out_cp.wait()

        def group_body(g, carry, buf=buf):
            rowv = g * L + lane
            for p in range(PASSES):
                def n_body(n, accs, p=p):
                    idxv = inv_bufs[buf][n, pl.ds(g * L, L)]
                    bl = ((idxv * WORDS) ^ lane) + n * (K * WORDS)
                    accs = list(accs)
                    for w8 in range(WPP):
                        word = plsc.load_gather(tbl_v, [bl ^ (p * WPP + w8)])
                        accs[2 * w8] = accs[2 * w8] + plsc.bitcast(
                            word << 16, jnp.float32)
                        accs[2 * w8 + 1] = accs[2 * w8 + 1] + plsc.bitcast(
                            word & hi_mask, jnp.float32)
                    return accs

                accs = lax.fori_loop(
                    0, N, n_body, [jnp.zeros((L,), jnp.float32)] * L,
                    unroll=2)
                rowv64 = rowv * OUT
                for w8 in range(WPP):
                    col = ((p * WPP + w8) ^ lane) * 2
                    plsc.store_scatter(outb_v, [rowv64 + col], accs[2 * w8])
                    plsc.store_scatter(outb_v, [rowv64 + col + 1],
                                       accs[2 * w8 + 1])
            return carry

        lax.fori_loop(0, GROUPS, group_body, 0)
        row0 = rowbase + ch * CHUNK
        out_cp = pltpu.async_copy(
            outb_v, out_hbm.at[pl.ds(row0 * OUT, CHUNK * OUT)], o_sem)

    out_cp.wait()


def kernel(inventory, node_embeds, W, b):
    tbl16 = _tc_table(node_embeds, W, b.reshape(1, OUT))
    tbl = lax.bitcast_convert_type(
        tbl16.reshape(N * K, WORDS, 2), jnp.int32).reshape(-1)
    out_flat = _sc_gather_sum(tbl, inventory.T)
    return out_flat.reshape(B, OUT)


# speedup vs baseline: 1.1054x; 1.1054x over previous
import functools

import jax
import jax.numpy as jnp
from jax import lax
from jax.experimental import pallas as pl
from jax.experimental.pallas import tpu as pltpu
from jax.experimental.pallas import tpu_sc as plsc

B = 16384
N = 50
D = 64
K = 64
OUT = 64

NC = 2
NS = 16
L = 16

WORDS = OUT // 2
ROWS_PER_TEC = B // (NC * NS)
CHUNK = 128
GROUPS = CHUNK // L
NCHUNK = ROWS_PER_TEC // CHUNK
PASSES = 4
WPP = WORDS // PASSES


def _tc_table(node_embeds, w, b2):
    def body(e_ref, w_ref, b_ref, o_ref):
        e = e_ref[...]
        bias = b_ref[...]
        for n in range(N):
            w_blk = w_ref[:, n * D:(n + 1) * D]
            p = lax.dot_general(e, w_blk, (((1,), (1,)), ((), ())),
                                preferred_element_type=jnp.float32)
            if n == 0:
                p = p + bias
            o_ref[pl.ds(n * K, K), :] = p.astype(jnp.bfloat16)

    return pl.pallas_call(
        body,
        out_shape=jax.ShapeDtypeStruct((N * K, OUT), jnp.bfloat16),
    )(node_embeds, w, b2)


_sc_mesh = plsc.VectorSubcoreMesh(
    core_axis_name="c", subcore_axis_name="s", num_cores=NC, num_subcores=NS)


@functools.partial(
    pl.kernel,
    out_type=jax.ShapeDtypeStruct((OUT, B), jnp.float32),
    mesh=_sc_mesh,
    compiler_params=pltpu.CompilerParams(needs_layout_passes=False),
    scratch_types=[
        pltpu.VMEM((N * K * WORDS,), jnp.int32),
        pltpu.VMEM((N, CHUNK), jnp.int32),
        pltpu.VMEM((N, CHUNK), jnp.int32),
        pltpu.VMEM((OUT, CHUNK), jnp.float32),
        pltpu.SemaphoreType.DMA,
        pltpu.SemaphoreType.DMA,
        pltpu.SemaphoreType.DMA,
        pltpu.SemaphoreType.DMA,
    ],
)
def _sc_gather_sum(tbl_hbm, invt_hbm, out_hbm,
                   tbl_v, inv_v0, inv_v1, outb_v,
                   tbl_sem, i_sem0, i_sem1, o_sem):
    cid = lax.axis_index("c")
    sid = lax.axis_index("s")
    wid = sid * NC + cid
    rowbase = wid * ROWS_PER_TEC
    inv_bufs = (inv_v0, inv_v1)
    i_sems = (i_sem0, i_sem1)

    tbl_cp = pltpu.async_copy(tbl_hbm, tbl_v, tbl_sem)

    def inv_copy(ch):
        row0 = rowbase + ch * CHUNK
        return pltpu.async_copy(
            invt_hbm.at[:, pl.ds(row0, CHUNK)], inv_bufs[ch % 2],
            i_sems[ch % 2])

    inv_cps = [inv_copy(0)]
    tbl_cp.wait()

    lane = lax.iota(jnp.int32, L)
    hi_mask = jnp.full((L,), -65536, jnp.int32)
    out_cp = None

    for ch in range(NCHUNK):
        buf = ch % 2
        inv_cps[ch].wait()
        if ch + 1 < NCHUNK:
            inv_cps.append(inv_copy(ch + 1))
        if out_cp is not None:
            out_cp.wait()

        def group_body(g, carry, buf=buf):
            rowv = g * L + lane
            for p in range(PASSES):
                def n_body(n, accs, p=p):
                    idxv = inv_bufs[buf][n, pl.ds(g * L, L)]
                    bl = ((idxv * WORDS) ^ lane) + n * (K * WORDS)
                    accs = list(accs)
                    for w8 in range(WPP):
                        word = plsc.load_gather(tbl_v, [bl ^ (p * WPP + w8)])
                        accs[2 * w8] = accs[2 * w8] + plsc.bitcast(
                            word << 16, jnp.float32)
                        accs[2 * w8 + 1] = accs[2 * w8 + 1] + plsc.bitcast(
                            word & hi_mask, jnp.float32)
                    return accs

                accs = lax.fori_loop(
                    0, N, n_body, [jnp.zeros((L,), jnp.float32)] * L,
                    unroll=5)
                for w8 in range(WPP):
                    col = ((p * WPP + w8) ^ lane) * 2
                    plsc.store_scatter(outb_v, [col, rowv], accs[2 * w8])
                    plsc.store_scatter(outb_v, [col + 1, rowv],
                                       accs[2 * w8 + 1])
            return carry

        lax.fori_loop(0, GROUPS, group_body, 0)
        row0 = rowbase + ch * CHUNK
        out_cp = pltpu.async_copy(
            outb_v, out_hbm.at[:, pl.ds(row0, CHUNK)], o_sem)

    out_cp.wait()


def kernel(inventory, node_embeds, W, b):
    tbl16 = _tc_table(node_embeds, W, b.reshape(1, OUT))
    tbl = lax.bitcast_convert_type(
        tbl16.reshape(N * K, WORDS, 2), jnp.int32).reshape(-1)
    out_t = _sc_gather_sum(tbl, inventory.T)
    return out_t.T


# speedup vs baseline: 1.1873x; 1.0741x over previous
import functools

import jax
import jax.numpy as jnp
from jax import lax
from jax.experimental import pallas as pl
from jax.experimental.pallas import tpu as pltpu
from jax.experimental.pallas import tpu_sc as plsc

B = 16384
N = 50
D = 64
K = 64
OUT = 64

NC = 2
NS = 16
L = 16

WORDS = OUT // 2
ROWS_PER_TEC = B // (NC * NS)
CHUNK = 128
GROUPS = CHUNK // L
NCHUNK = ROWS_PER_TEC // CHUNK
PASSES = 2
WPP = WORDS // PASSES


def _tc_table(node_embeds, w, b2):
    def body(e_ref, w_ref, b_ref, o_ref):
        e = e_ref[...]
        bias = b_ref[...]
        for n in range(N):
            w_blk = w_ref[:, n * D:(n + 1) * D]
            p = lax.dot_general(e, w_blk, (((1,), (1,)), ((), ())),
                                preferred_element_type=jnp.float32)
            if n == 0:
                p = p + bias
            o_ref[pl.ds(n * K, K), :] = p.astype(jnp.bfloat16)

    return pl.pallas_call(
        body,
        out_shape=jax.ShapeDtypeStruct((N * K, OUT), jnp.bfloat16),
    )(node_embeds, w, b2)


_sc_mesh = plsc.VectorSubcoreMesh(
    core_axis_name="c", subcore_axis_name="s", num_cores=NC, num_subcores=NS)


@functools.partial(
    pl.kernel,
    out_type=jax.ShapeDtypeStruct((OUT, B), jnp.float32),
    mesh=_sc_mesh,
    compiler_params=pltpu.CompilerParams(needs_layout_passes=False),
    scratch_types=[
        pltpu.VMEM((N * K * WORDS,), jnp.int32),
        pltpu.VMEM((N, CHUNK), jnp.int32),
        pltpu.VMEM((N, CHUNK), jnp.int32),
        pltpu.VMEM((OUT, CHUNK), jnp.float32),
        pltpu.SemaphoreType.DMA,
        pltpu.SemaphoreType.DMA,
        pltpu.SemaphoreType.DMA,
        pltpu.SemaphoreType.DMA,
    ],
)
def _sc_gather_sum(tbl_hbm, invt_hbm, out_hbm,
                   tbl_v, inv_v0, inv_v1, outb_v,
                   tbl_sem, i_sem0, i_sem1, o_sem):
    cid = lax.axis_index("c")
    sid = lax.axis_index("s")
    wid = sid * NC + cid
    rowbase = wid * ROWS_PER_TEC
    inv_bufs = (inv_v0, inv_v1)
    i_sems = (i_sem0, i_sem1)

    tbl_cp = pltpu.async_copy(tbl_hbm, tbl_v, tbl_sem)

    def inv_copy(ch):
        row0 = rowbase + ch * CHUNK
        return pltpu.async_copy(
            invt_hbm.at[:, pl.ds(row0, CHUNK)], inv_bufs[ch % 2],
            i_sems[ch % 2])

    inv_cps = [inv_copy(0)]
    tbl_cp.wait()

    lane = lax.iota(jnp.int32, L)
    hi_mask = jnp.full((L,), -65536, jnp.int32)
    out_cp = None

    for ch in range(NCHUNK):
        buf = ch % 2
        inv_cps[ch].wait()
        if ch + 1 < NCHUNK:
            inv_cps.append(inv_copy(ch + 1))
        if out_cp is not None:
            out_cp.wait()

        def group_body(g, carry, buf=buf):
            rowv = g * L + lane
            for p in range(PASSES):
                def n_body(n, accs, p=p):
                    idxv = inv_bufs[buf][n, pl.ds(g * L, L)]
                    bl = ((idxv * WORDS) ^ lane) + n * (K * WORDS)
                    accs = list(accs)
                    for w8 in range(WPP):
                        word = plsc.load_gather(tbl_v, [bl ^ (p * WPP + w8)])
                        accs[2 * w8] = accs[2 * w8] + plsc.bitcast(
                            word << 16, jnp.float32)
                        accs[2 * w8 + 1] = accs[2 * w8 + 1] + plsc.bitcast(
                            word & hi_mask, jnp.float32)
                    return accs

                accs = lax.fori_loop(
                    0, N, n_body,
                    [jnp.zeros((L,), jnp.float32)] * (2 * WPP),
                    unroll=1)
                for w8 in range(WPP):
                    col = ((p * WPP + w8) ^ lane) * 2
                    plsc.store_scatter(outb_v, [col, rowv], accs[2 * w8])
                    plsc.store_scatter(outb_v, [col + 1, rowv],
                                       accs[2 * w8 + 1])
            return carry

        lax.fori_loop(0, GROUPS, group_body, 0)
        row0 = rowbase + ch * CHUNK
        out_cp = pltpu.async_copy(
            outb_v, out_hbm.at[:, pl.ds(row0, CHUNK)], o_sem)

    out_cp.wait()


def kernel(inventory, node_embeds, W, b):
    tbl16 = _tc_table(node_embeds, W, b.reshape(1, OUT))
    tbl = lax.bitcast_convert_type(
        tbl16.reshape(N * K, WORDS, 2), jnp.int32).reshape(-1)
    out_t = _sc_gather_sum(tbl, inventory.T)
    return out_t.T
